# Initial kernel scaffold; baseline (speedup 1.0000x reference)
import jax
import jax.numpy as jnp
from jax.experimental import pallas as pl


def kernel(x, max_index, neigh_orders):
    raise NotImplementedError("write your pallas kernel here")


# speedup vs baseline: 7.7888x; 7.7888x over previous
import functools
import jax
import jax.numpy as jnp
from jax import lax
from jax.experimental import pallas as pl
from jax.experimental.pallas import tpu as pltpu
from jax.experimental.pallas import tpu_sc as plsc

RAW = 40962
FEAT = 128
NNODES = 163842
N = RAW * FEAT
NM1 = N - 1
OUT = NNODES * FEAT
PADROWS = 40976
PADX = PADROWS * FEAT
PADNO = 286_760
ROWS = 322
ROWSL = 336
BUFL = ROWS * FEAT
NOB = 2272
ZCH = 7896
ZREP = 166
STRIPE = OUT // 16

_mesh = plsc.VectorSubcoreMesh(core_axis_name="c", subcore_axis_name="s")


@functools.partial(
    pl.kernel,
    out_type=jax.ShapeDtypeStruct((OUT,), jnp.float32),
    mesh=_mesh,
    scratch_types=[
        pltpu.VMEM((BUFL + 16,), jnp.float32),
        pltpu.VMEM((ROWSL, FEAT), jnp.int32),
        pltpu.VMEM((NOB,), jnp.int32),
        pltpu.VMEM((ZCH + 8,), jnp.float32),
        pltpu.VMEM((32,), jnp.int32),
        pltpu.SemaphoreType.DMA,
    ],
)
def _sck(xp, mip, nop, out, x_buf, mi2, no_buf, zbuf, tmp, sem):
    cid = lax.axis_index("c")
    sid = lax.axis_index("s")
    lanes = lax.iota(jnp.int32, 16)
    active = (cid == 0).astype(jnp.int32)

    def zfill(u, _):
        zbuf[pl.ds(16 * u, 16)] = jnp.zeros((16,), jnp.float32)
        return 0
    lax.fori_loop(0, (ZCH + 8) // 16, zfill, 0)
    zstart = sid * STRIPE

    def zfire(d, _):
        pltpu.async_copy(
            zbuf.at[pl.ds(0, ZCH)], out.at[pl.ds(pl.multiple_of(zstart + ZCH * d, 8), ZCH)], sem)
        return 0
    lax.fori_loop(0, ZREP * active, zfire, 0)

    def zdrain(d, _):
        pltpu.make_async_copy(
            zbuf.at[pl.ds(0, ZCH)], out.at[pl.ds(zstart, ZCH)], sem).wait()
        return 0
    lax.fori_loop(0, ZREP * active, zdrain, 0)

    plsc.subcore_barrier()

    neg1 = jnp.full((16,), -1, jnp.int32)

    def do_col(q, _):
        c = sid * 8 + q
        ks = (c * NM1 + 127) >> 7
        ke = (((c + 1) * NM1 + 127) >> 7) + jnp.where(c == 127, 1, 0)
        is_ = ks >> 7
        base = pl.multiple_of(is_ << 7, 128)
        nob = pl.multiple_of((7 * is_) & ~7, 8)
        is8 = pl.multiple_of(is_ & ~7, 8)
        roff = is_ - is8
        nrows = (ke - base + 127) >> 7

        pltpu.sync_copy(xp.at[pl.ds(base, BUFL)], x_buf.at[pl.ds(0, BUFL)])
        pltpu.sync_copy(mip.at[pl.ds(is8, ROWSL), :], mi2)
        pltpu.sync_copy(nop.at[pl.ds(nob, NOB)], no_buf)
        tmp[pl.ds(16, 16)] = neg1

        def row_body(r, _):
            nv = no_buf[pl.ds(7 * (is_ + r) - nob, 16)]
            sd = [(nv[m] << 7) + c for m in range(7)]
            run = [neg1] * 7
            for v in range(8):
                tt = (r << 7) + 16 * v
                mi_v = mi2[roff + r, pl.ds(16 * v, 16)]
                kv = base + tt + lanes
                valid = jnp.logical_and(kv >= ks, kv < ke)
                tvec = tt + lanes
                for m in range(7):
                    hit = jnp.logical_and(valid, mi_v == m)
                    run[m] = jnp.maximum(run[m], jnp.where(hit, tvec, -1))
            win = []
            for m in range(7):
                tmp[pl.ds(0, 16)] = run[m]
                r1 = jnp.maximum(run[m], tmp[pl.ds(8, 16)])
                tmp[pl.ds(0, 16)] = r1
                r2 = jnp.maximum(r1, tmp[pl.ds(4, 16)])
                tmp[pl.ds(0, 16)] = r2
                r3 = jnp.maximum(r2, tmp[pl.ds(2, 16)])
                tmp[pl.ds(0, 16)] = r3
                r4 = jnp.maximum(r3, tmp[pl.ds(1, 16)])
                win.append(r4[0])
            val = [x_buf[pl.ds(jnp.maximum(win[m], 0), 16)][0] for m in range(7)]
            bval = []
            for a in range(7):
                bw, bv = win[a], val[a]
                for b in range(7):
                    if b == a:
                        continue
                    upd = jnp.logical_and(sd[b] == sd[a], win[b] > bw)
                    bv = jnp.where(upd, val[b], bv)
                    bw = jnp.where(upd, win[b], bw)
                bval.append(bv)
            dv = sd[0]
            vv = bval[0]
            for m in range(1, 7):
                ok = win[m] >= 0
                dv = jnp.where(ok, sd[m], dv)
                vv = jnp.where(ok, bval[m], vv)
            for v in range(8):
                tt = (r << 7) + 16 * v
                mi_v = mi2[roff + r, pl.ds(16 * v, 16)]
                kv = base + tt + lanes
                d = jnp.full((16,), 0, jnp.int32) + sd[6]
                xw = jnp.full((16,), 0.0, jnp.float32) + bval[6]
                for m in range(5, -1, -1):
                    d = jnp.where(mi_v == m, sd[m], d)
                    xw = jnp.where(mi_v == m, bval[m], xw)
                outside = jnp.logical_or(kv < ks, kv >= ke)
                d = jnp.where(outside, dv, d)
                xw = jnp.where(outside, vv, xw)
                mi2[roff + r, pl.ds(16 * v, 16)] = d
                x_buf[pl.ds(pl.multiple_of(tt, 16), 16)] = xw
            pltpu.async_copy(
                x_buf.at[pl.ds(pl.multiple_of(r * 128, 128), 128)],
                out.at[mi2.at[roff + r]], sem)
            return 0
        lax.fori_loop(0, nrows, row_body, 0)

        def drain(r, _):
            pltpu.make_async_copy(
                x_buf.at[pl.ds(0, 128)], out.at[mi2.at[0]], sem).wait()
            return 0
        lax.fori_loop(0, nrows, drain, 0)
        return 0

    lax.fori_loop(0, 8 * active, do_col, 0)


@jax.jit
def kernel(x, max_index, neigh_orders):
    xf = x.reshape(-1)
    xpad = jnp.concatenate([xf, jnp.zeros((PADX - N,), xf.dtype)])
    mif = max_index.reshape(-1).astype(jnp.int32)
    mip = jnp.concatenate(
        [mif, jnp.zeros((PADX - N,), jnp.int32)]).reshape(PADROWS, FEAT)
    no = neigh_orders.astype(jnp.int32)
    nop = jnp.concatenate(
        [no, jnp.zeros((PADNO - no.shape[0],), jnp.int32)])
    y = _sck(xpad, mip, nop)
    return y.reshape(NNODES, FEAT)
